# baseline (speedup 1.0000x reference)
import functools

import jax
import jax.numpy as jnp
from jax import lax
from jax.experimental import pallas as pl
from jax.experimental.pallas import tpu as pltpu
from jax.experimental.pallas import tpu_sc as plsc

_NC = 2
_NS = 16
_LANES = 16
_NW = _NC * _NS
_LN2 = 0.6931471805599453


@functools.partial(jax.jit, static_argnums=(3,))
def _sc_lr_unigram(x, W, b, NG):
    L, B = x.shape
    V = W.shape[1]
    assert NG == 2, "kernel body hand-unrolls exactly two 16-batch groups"
    bw = NG * _LANES
    mesh = plsc.VectorSubcoreMesh(core_axis_name="c", subcore_axis_name="s",
                                  num_cores=_NC, num_subcores=_NS)

    @functools.partial(
        pl.kernel,
        out_type=jax.ShapeDtypeStruct((2 * B,), jnp.float32),
        mesh=mesh,
        scratch_types=[
            pltpu.VMEM((L, 128), jnp.int32),
            pltpu.VMEM((V,), jnp.float32),
            pltpu.VMEM((V,), jnp.float32),
            pltpu.VMEM((2 * _LANES,), jnp.float32),
            pltpu.VMEM((bw,), jnp.float32),
            pltpu.VMEM((bw,), jnp.float32),
            pltpu.SemaphoreType.DMA,
        ],
        compiler_params=pltpu.CompilerParams(needs_layout_passes=False,
                                             skip_device_barrier=True),
    )
    def k(x_hbm, w_hbm, b_hbm, o_hbm, idx_v, w0_v, w1_v, b_v, o0_v, o1_v,
          sem):
        wid = lax.axis_index("s") * _NC + lax.axis_index("c")
        base = wid * bw
        blk = pl.multiple_of((wid // 4) * 128, 128)
        cp = pltpu.async_copy(x_hbm.at[:, pl.ds(blk, 128)], idx_v, sem)
        cw0 = pltpu.async_copy(w_hbm.at[0], w0_v, sem)
        cw1 = pltpu.async_copy(w_hbm.at[1], w1_v, sem)
        cb = pltpu.async_copy(b_hbm, b_v, sem)
        cp.wait()
        cw0.wait()
        cw1.wait()
        cb.wait()

        b0 = b_v[pl.ds(0, _LANES)]
        b1 = b_v[pl.ds(_LANES, _LANES)]
        col = (wid % 4) * bw + jax.lax.iota(jnp.int32, 16)

        UNROLL = 5
        assert L % UNROLL == 0

        def acc(j, carry):
            a00, a10, a01, a11 = carry
            for u in range(UNROLL):
                row = jnp.full((_LANES,), j * UNROLL + u, jnp.int32)
                t0 = plsc.load_gather(idx_v, [row, col])
                t1 = plsc.load_gather(idx_v, [row, col + _LANES])
                a00 = a00 + plsc.load_gather(w0_v, [t0])
                a10 = a10 + plsc.load_gather(w1_v, [t0])
                a01 = a01 + plsc.load_gather(w0_v, [t1])
                a11 = a11 + plsc.load_gather(w1_v, [t1])
            return (a00, a10, a01, a11)
        a00, a10, a01, a11 = lax.fori_loop(0, L // UNROLL, acc,
                                           (b0, b1, b0, b1))

        for g, (az, ao) in enumerate(((a00, a10), (a01, a11))):
            s0 = 1.0 / (1.0 + jnp.exp(-az))
            s1 = 1.0 / (1.0 + jnp.exp(-ao))
            d = s1 - s0
            u2 = 0.25 * d * d
            sp = 0.5 * d + _LN2 + u2 * (0.5 + u2 * (-1.0 / 12.0
                                                    + u2 * (1.0 / 45.0)))
            sp = sp + (1.0 + jnp.exp(d)) * jnp.exp(-sp) - 1.0
            o0_v[pl.ds(g * _LANES, _LANES)] = -sp
            o1_v[pl.ds(g * _LANES, _LANES)] = d - sp
        pltpu.sync_copy(o0_v, o_hbm.at[pl.ds(base, bw)])
        pltpu.sync_copy(o1_v, o_hbm.at[pl.ds(B + base, bw)])

    b16 = jnp.broadcast_to(b[:, None], (2, _LANES)).reshape(2 * _LANES)
    return k(x, W, b16)


def kernel(x, embed_weight, W, b):
    L, B = x.shape
    del embed_weight
    NG = B // (_NW * _LANES)
    oflat = _sc_lr_unigram(x, W, b, NG)
    return oflat.reshape(2, B).T


# speedup vs baseline: 1.0621x; 1.0621x over previous
import functools

import jax
import jax.numpy as jnp
from jax import lax
from jax.experimental import pallas as pl
from jax.experimental.pallas import tpu as pltpu
from jax.experimental.pallas import tpu_sc as plsc

_NC = 2
_NS = 16
_LANES = 16
_NW = _NC * _NS
_LN2 = 0.6931471805599453


@functools.partial(jax.jit, static_argnums=(3,))
def _sc_lr_unigram(x, W, b, NG):
    L, B = x.shape
    V = W.shape[1]
    assert NG == 2, "kernel body hand-unrolls exactly two 16-batch groups"
    bw = NG * _LANES
    mesh = plsc.VectorSubcoreMesh(core_axis_name="c", subcore_axis_name="s",
                                  num_cores=_NC, num_subcores=_NS)

    @functools.partial(
        pl.kernel,
        out_type=jax.ShapeDtypeStruct((2, B), jnp.float32),
        mesh=mesh,
        scratch_types=[
            pltpu.VMEM((L, 128), jnp.int32),
            pltpu.VMEM((V,), jnp.float32),
            pltpu.VMEM((V,), jnp.float32),
            pltpu.VMEM((2 * _LANES,), jnp.float32),
            pltpu.VMEM((bw,), jnp.float32),
            pltpu.VMEM((bw,), jnp.float32),
            pltpu.SemaphoreType.DMA,
        ],
        compiler_params=pltpu.CompilerParams(needs_layout_passes=False),
    )
    def k(x_hbm, w_hbm, b_hbm, o_hbm, idx_v, w0_v, w1_v, b_v, o0_v, o1_v,
          sem):
        wid = lax.axis_index("s") * _NC + lax.axis_index("c")
        base = wid * bw
        blk = pl.multiple_of((wid // 4) * 128, 128)
        cp = pltpu.async_copy(x_hbm.at[:, pl.ds(blk, 128)], idx_v, sem)
        cw0 = pltpu.async_copy(w_hbm.at[0], w0_v, sem)
        cw1 = pltpu.async_copy(w_hbm.at[1], w1_v, sem)
        cb = pltpu.async_copy(b_hbm, b_v, sem)
        cp.wait()
        cw0.wait()
        cw1.wait()
        cb.wait()

        b0 = b_v[pl.ds(0, _LANES)]
        b1 = b_v[pl.ds(_LANES, _LANES)]
        col = (wid % 4) * bw + jax.lax.iota(jnp.int32, 16)

        UNROLL = 5
        assert L % UNROLL == 0

        def acc(j, carry):
            a00, a10, a01, a11 = carry
            for u in range(UNROLL):
                row = jnp.full((_LANES,), j * UNROLL + u, jnp.int32)
                t0 = plsc.load_gather(idx_v, [row, col])
                t1 = plsc.load_gather(idx_v, [row, col + _LANES])
                a00 = a00 + plsc.load_gather(w0_v, [t0])
                a10 = a10 + plsc.load_gather(w1_v, [t0])
                a01 = a01 + plsc.load_gather(w0_v, [t1])
                a11 = a11 + plsc.load_gather(w1_v, [t1])
            return (a00, a10, a01, a11)
        a00, a10, a01, a11 = lax.fori_loop(0, L // UNROLL, acc,
                                           (b0, b1, b0, b1))

        for g, (az, ao) in enumerate(((a00, a10), (a01, a11))):
            s0 = 1.0 / (1.0 + jnp.exp(-az))
            s1 = 1.0 / (1.0 + jnp.exp(-ao))
            d = s1 - s0
            u2 = 0.25 * d * d
            sp = 0.5 * d + _LN2 + u2 * (0.5 + u2 * (-1.0 / 12.0
                                                    + u2 * (1.0 / 45.0)))
            sp = sp + (1.0 + jnp.exp(d)) * jnp.exp(-sp) - 1.0
            o0_v[pl.ds(g * _LANES, _LANES)] = -sp
            o1_v[pl.ds(g * _LANES, _LANES)] = d - sp
        pltpu.sync_copy(o0_v, o_hbm.at[0, pl.ds(base, bw)])
        pltpu.sync_copy(o1_v, o_hbm.at[1, pl.ds(base, bw)])

    b16 = jnp.broadcast_to(b[:, None], (2, _LANES)).reshape(2 * _LANES)
    return k(x, W, b16)


def kernel(x, embed_weight, W, b):
    L, B = x.shape
    del embed_weight
    NG = B // (_NW * _LANES)
    out2 = _sc_lr_unigram(x, W, b, NG)
    return out2.T
